# baseline (speedup 1.0000x reference)
import functools

import jax
import jax.numpy as jnp
from jax import lax
from jax.experimental import pallas as pl
from jax.experimental.pallas import tpu as pltpu
from jax.experimental.pallas import tpu_sc as plsc

EMB = 32
B_TOTAL = 4096 * 200
NUM_WORKERS = 32
TOK = 1024
UNITS = B_TOTAL // TOK
UPW = UNITS // NUM_WORKERS


def _emb_body(x_hbm, table_hbm, out_hbm, idx0, idx1, buf0, buf1, slab,
              semg0, semg1):
    wid = lax.axis_index("s") * 2 + lax.axis_index("c")
    base_u = wid * UPW

    iota16 = lax.iota(jnp.int32, 16)

    def load_idx(u, idx_v):
        pltpu.sync_copy(x_hbm.at[pl.ds(u * TOK, TOK)], idx_v)

    def start_gather(idx_v, buf, semg):
        return pltpu.async_copy(table_hbm.at[idx_v], buf, semg)

    def transpose_unit(buf):
        @plsc.parallel_loop(0, 64, unroll=2)
        def body(i):
            c = lax.shift_right_logical(i, 3)
            m = lax.bitwise_and(i, 7)
            rows = iota16 + (c * 128 + m * 16)
            dl = m * 16
            for jj in range(32):
                cols = jnp.full((16,), jj, jnp.int32)
                v = plsc.load_gather(buf, [rows, cols])
                slab[jj // 8, c, jj % 8, pl.ds(dl, 16)] = v

    def write_slab(u):
        s = u // 4
        q = u % 4
        pltpu.sync_copy(
            slab, out_hbm.at[s, slice(None), pl.ds(q * 8, 8)])

    load_idx(base_u, idx0)
    g0 = start_gather(idx0, buf0, semg0)

    def pair(i, carry):
        u0 = base_u + 2 * i
        load_idx(u0 + 1, idx1)
        start_gather(idx1, buf1, semg1)
        pltpu.make_async_copy(table_hbm.at[idx0], buf0, semg0).wait()
        transpose_unit(buf0)
        write_slab(u0)
        load_idx(u0 + 2, idx0)
        start_gather(idx0, buf0, semg0)
        pltpu.make_async_copy(table_hbm.at[idx1], buf1, semg1).wait()
        transpose_unit(buf1)
        write_slab(u0 + 1)
        return carry

    lax.fori_loop(0, (UPW - 1) // 2, pair, 0)

    u_last = base_u + UPW - 1
    pltpu.make_async_copy(table_hbm.at[idx0], buf0, semg0).wait()
    transpose_unit(buf0)
    write_slab(u_last)


def kernel(x, table):
    xf = x.T.reshape(-1)
    mesh = plsc.VectorSubcoreMesh(core_axis_name="c", subcore_axis_name="s")
    run = functools.partial(
        pl.kernel,
        mesh=mesh,
        out_type=jax.ShapeDtypeStruct((200, 4, 32, 8, 128), jnp.float32),
        scratch_types=[
            pltpu.VMEM((TOK,), jnp.int32),
            pltpu.VMEM((TOK,), jnp.int32),
            pltpu.VMEM((TOK, EMB), jnp.float32),
            pltpu.VMEM((TOK, EMB), jnp.float32),
            pltpu.VMEM((4, 8, 8, 128), jnp.float32),
            pltpu.SemaphoreType.DMA,
            pltpu.SemaphoreType.DMA,
        ],
        compiler_params=pltpu.CompilerParams(
            use_tc_tiling_on_sc=False, needs_layout_passes=False),
    )(_emb_body)
    out5d = run(xf, table)
    return out5d.transpose(2, 4, 0, 1, 3).reshape(4096, 200, 32)


# speedup vs baseline: 1.0436x; 1.0436x over previous
import functools

import jax
import jax.numpy as jnp
from jax import lax
from jax.experimental import pallas as pl
from jax.experimental.pallas import tpu as pltpu
from jax.experimental.pallas import tpu_sc as plsc

EMB = 32
B_TOTAL = 4096 * 200
NUM_WORKERS = 32
TOK = 1024
UNITS = B_TOTAL // TOK
UPW = UNITS // NUM_WORKERS


def _emb_body(x_hbm, table_hbm, out_hbm, idx0, idx1, buf0, buf1, slab,
              semg0, semg1):
    wid = lax.axis_index("s") * 2 + lax.axis_index("c")
    base_u = wid * UPW

    iota16 = lax.iota(jnp.int32, 16)

    def load_idx(u, idx_v):
        pltpu.sync_copy(x_hbm.at[pl.ds(u * TOK, TOK)], idx_v)

    def start_gather(idx_v, buf, semg):
        return pltpu.async_copy(table_hbm.at[idx_v], buf, semg)

    def transpose_unit(buf):
        @plsc.parallel_loop(0, 256, unroll=8)
        def body(i):
            c = lax.shift_right_logical(i, 5)
            jj = lax.bitwise_and(i, 31)
            a = lax.shift_right_logical(jj, 3)
            r = lax.bitwise_and(jj, 7)
            cols = jnp.full((16,), jj, jnp.int32)
            base = iota16 + c * 128
            for m in range(8):
                v = plsc.load_gather(buf, [base + m * 16, cols])
                slab[a, c, r, pl.ds(m * 16, 16)] = v

    def write_slab(u):
        s = u // 4
        q = u % 4
        pltpu.sync_copy(
            slab, out_hbm.at[s, slice(None), pl.ds(q * 8, 8)])

    load_idx(base_u, idx0)
    g0 = start_gather(idx0, buf0, semg0)

    def pair(i, carry):
        u0 = base_u + 2 * i
        load_idx(u0 + 1, idx1)
        start_gather(idx1, buf1, semg1)
        pltpu.make_async_copy(table_hbm.at[idx0], buf0, semg0).wait()
        transpose_unit(buf0)
        write_slab(u0)
        load_idx(u0 + 2, idx0)
        start_gather(idx0, buf0, semg0)
        pltpu.make_async_copy(table_hbm.at[idx1], buf1, semg1).wait()
        transpose_unit(buf1)
        write_slab(u0 + 1)
        return carry

    lax.fori_loop(0, (UPW - 1) // 2, pair, 0)

    u_last = base_u + UPW - 1
    pltpu.make_async_copy(table_hbm.at[idx0], buf0, semg0).wait()
    transpose_unit(buf0)
    write_slab(u_last)


def kernel(x, table):
    xf = x.T.reshape(-1)
    mesh = plsc.VectorSubcoreMesh(core_axis_name="c", subcore_axis_name="s")
    run = functools.partial(
        pl.kernel,
        mesh=mesh,
        out_type=jax.ShapeDtypeStruct((200, 4, 32, 8, 128), jnp.float32),
        scratch_types=[
            pltpu.VMEM((TOK,), jnp.int32),
            pltpu.VMEM((TOK,), jnp.int32),
            pltpu.VMEM((TOK, EMB), jnp.float32),
            pltpu.VMEM((TOK, EMB), jnp.float32),
            pltpu.VMEM((4, 8, 8, 128), jnp.float32),
            pltpu.SemaphoreType.DMA,
            pltpu.SemaphoreType.DMA,
        ],
        compiler_params=pltpu.CompilerParams(
            use_tc_tiling_on_sc=False, needs_layout_passes=False),
    )(_emb_body)
    out5d = run(xf, table)
    return out5d.transpose(2, 4, 0, 1, 3).reshape(4096, 200, 32)


# speedup vs baseline: 1.5193x; 1.4558x over previous
import functools

import jax
import jax.numpy as jnp
from jax import lax
from jax.experimental import pallas as pl
from jax.experimental.pallas import tpu as pltpu
from jax.experimental.pallas import tpu_sc as plsc

EMB = 32
B_TOTAL = 4096 * 200
NUM_WORKERS = 32
TOK = 1024
UNITS = B_TOTAL // TOK
UPW = UNITS // NUM_WORKERS


def _emb_body(x_hbm, table_hbm, out_hbm, idx0, idx1, buf0, buf1, slab,
              semg0, semg1):
    wid = lax.axis_index("s") * 2 + lax.axis_index("c")
    base_u = wid * UPW

    iota16 = lax.iota(jnp.int32, 16)

    def load_idx(u, idx_v):
        pltpu.sync_copy(x_hbm.at[pl.ds(u * TOK, TOK)], idx_v)

    def start_gather(idx_v, buf, semg):
        return pltpu.async_copy(table_hbm.at[idx_v], buf, semg)

    a_lo = lax.shift_right_logical(iota16, 3)
    a_hi = a_lo + 2
    r_v = lax.bitwise_and(iota16, 7)

    def transpose_unit(buf):
        @plsc.parallel_loop(0, TOK, unroll=4)
        def body(t):
            c = lax.shift_right_logical(t, 7)
            l = lax.bitwise_and(t, 127)
            cv = jnp.full((16,), c, jnp.int32)
            lv = jnp.full((16,), l, jnp.int32)
            v0 = buf[t, pl.ds(0, 16)]
            plsc.store_scatter(slab, [a_lo, cv, r_v, lv], v0)
            v1 = buf[t, pl.ds(16, 16)]
            plsc.store_scatter(slab, [a_hi, cv, r_v, lv], v1)

    def write_slab(u):
        s = u // 4
        q = u % 4
        pltpu.sync_copy(
            slab.at[:, :, :, pl.ds(0, 128)],
            out_hbm.at[s, slice(None), pl.ds(q * 8, 8)])

    load_idx(base_u, idx0)
    g0 = start_gather(idx0, buf0, semg0)

    def wait_gather(idx_v, buf, semg):
        pltpu.make_async_copy(table_hbm.at[idx_v], buf, semg).wait()

    def pair(i, carry):
        u0 = base_u + 2 * i
        load_idx(u0 + 1, idx1)
        start_gather(idx1, buf1, semg1)
        wait_gather(idx0, buf0, semg0)
        transpose_unit(buf0)
        write_slab(u0)
        load_idx(u0 + 2, idx0)
        start_gather(idx0, buf0, semg0)
        wait_gather(idx1, buf1, semg1)
        transpose_unit(buf1)
        write_slab(u0 + 1)
        return carry

    lax.fori_loop(0, (UPW - 1) // 2, pair, 0)

    u_last = base_u + UPW - 1
    wait_gather(idx0, buf0, semg0)
    transpose_unit(buf0)
    write_slab(u_last)


def kernel(x, table):
    xf = x.T.reshape(-1)
    mesh = plsc.VectorSubcoreMesh(core_axis_name="c", subcore_axis_name="s")
    run = functools.partial(
        pl.kernel,
        mesh=mesh,
        out_type=jax.ShapeDtypeStruct((200, 4, 32, 8, 128), jnp.float32),
        scratch_types=[
            pltpu.VMEM((TOK,), jnp.int32),
            pltpu.VMEM((TOK,), jnp.int32),
            pltpu.VMEM((TOK, EMB), jnp.float32),
            pltpu.VMEM((TOK, EMB), jnp.float32),
            pltpu.VMEM((4, 8, 8, 132), jnp.float32),
            pltpu.SemaphoreType.DMA,
            pltpu.SemaphoreType.DMA,
        ],
        compiler_params=pltpu.CompilerParams(
            use_tc_tiling_on_sc=False, needs_layout_passes=False),
    )(_emb_body)
    out5d = run(xf, table)
    return out5d.transpose(2, 4, 0, 1, 3).reshape(4096, 200, 32)


# speedup vs baseline: 1.5210x; 1.0011x over previous
import functools

import jax
import jax.numpy as jnp
from jax import lax
from jax.experimental import pallas as pl
from jax.experimental.pallas import tpu as pltpu
from jax.experimental.pallas import tpu_sc as plsc

EMB = 32
B_TOTAL = 4096 * 200
NUM_WORKERS = 32
TOK = 1024
UNITS = B_TOTAL // TOK
UPW = UNITS // NUM_WORKERS


def _emb_body(x_hbm, table_hbm, out_hbm, idx0, idx1, buf0, buf1, slab,
              semg0, semg1):
    wid = lax.axis_index("s") * 2 + lax.axis_index("c")
    base_u = wid * UPW

    iota16 = lax.iota(jnp.int32, 16)

    def load_idx(u, idx_v):
        pltpu.sync_copy(x_hbm.at[pl.ds(u * TOK, TOK)], idx_v)

    def start_gather(idx_v, buf, semg):
        return pltpu.async_copy(table_hbm.at[idx_v], buf, semg)

    a_lo = lax.shift_right_logical(iota16, 3)
    a_hi = a_lo + 2
    r_v = lax.bitwise_and(iota16, 7)

    def transpose_unit(buf):
        @plsc.parallel_loop(0, TOK, unroll=4)
        def body(t):
            c = lax.shift_right_logical(t, 7)
            l = lax.bitwise_and(t, 127)
            cv = jnp.full((16,), c, jnp.int32)
            lv = jnp.full((16,), l, jnp.int32)
            v0 = buf[t, pl.ds(0, 16)]
            plsc.store_scatter(slab, [a_lo, cv, r_v, lv], v0)
            v1 = buf[t, pl.ds(16, 16)]
            plsc.store_scatter(slab, [a_hi, cv, r_v, lv], v1)

    def write_slab(u):
        s = u // 4
        q = u % 4
        pltpu.sync_copy(
            slab.at[:, :, :, pl.ds(0, 128)],
            out_hbm.at[s, slice(None), pl.ds(q * 8, 8)])

    load_idx(base_u, idx0)
    g0 = start_gather(idx0, buf0, semg0)

    def wait_gather(idx_v, buf, semg):
        pltpu.make_async_copy(table_hbm.at[idx_v], buf, semg).wait()

    def pair(i, carry):
        u0 = base_u + 2 * i
        load_idx(u0 + 1, idx1)
        start_gather(idx1, buf1, semg1)
        wait_gather(idx0, buf0, semg0)
        transpose_unit(buf0)
        write_slab(u0)
        load_idx(u0 + 2, idx0)
        start_gather(idx0, buf0, semg0)
        wait_gather(idx1, buf1, semg1)
        transpose_unit(buf1)
        write_slab(u0 + 1)
        return carry

    lax.fori_loop(0, (UPW - 1) // 2, pair, 0)

    u_last = base_u + UPW - 1
    wait_gather(idx0, buf0, semg0)
    transpose_unit(buf0)
    write_slab(u_last)


def kernel(x, table):
    xf = x.T.reshape(-1)
    mesh = plsc.VectorSubcoreMesh(core_axis_name="c", subcore_axis_name="s")
    run = functools.partial(
        pl.kernel,
        mesh=mesh,
        out_type=jax.ShapeDtypeStruct((200, 4, 32, 8, 128), jnp.float32),
        scratch_types=[
            pltpu.VMEM((TOK,), jnp.int32),
            pltpu.VMEM((TOK,), jnp.int32),
            pltpu.VMEM((TOK, EMB), jnp.float32),
            pltpu.VMEM((TOK, EMB), jnp.float32),
            pltpu.VMEM((4, 8, 8, 129), jnp.float32),
            pltpu.SemaphoreType.DMA,
            pltpu.SemaphoreType.DMA,
        ],
        compiler_params=pltpu.CompilerParams(
            use_tc_tiling_on_sc=False, needs_layout_passes=False),
    )(_emb_body)
    out5d = run(xf, table)
    return out5d.transpose(2, 4, 0, 1, 3).reshape(4096, 200, 32)


# speedup vs baseline: 1.5234x; 1.0016x over previous
import functools

import jax
import jax.numpy as jnp
from jax import lax
from jax.experimental import pallas as pl
from jax.experimental.pallas import tpu as pltpu
from jax.experimental.pallas import tpu_sc as plsc

EMB = 32
B_TOTAL = 4096 * 200
NUM_WORKERS = 32
TOK = 1024
UNITS = B_TOTAL // TOK
UPW = UNITS // NUM_WORKERS


def _emb_body(x_hbm, table_hbm, out_hbm, idx0, idx1, buf0, buf1, slab,
              semg0, semg1):
    wid = lax.axis_index("s") * 2 + lax.axis_index("c")
    base_u = wid * UPW

    iota16 = lax.iota(jnp.int32, 16)

    def load_idx(u, idx_v):
        pltpu.sync_copy(x_hbm.at[pl.ds(u * TOK, TOK)], idx_v)

    def start_gather(idx_v, buf, semg):
        return pltpu.async_copy(table_hbm.at[idx_v], buf, semg)

    a_lo = lax.shift_right_logical(iota16, 3)
    a_hi = a_lo + 2
    r_v = lax.bitwise_and(iota16, 7)

    def transpose_unit(buf):
        @plsc.parallel_loop(0, TOK, unroll=8)
        def body(t):
            c = lax.shift_right_logical(t, 7)
            l = lax.bitwise_and(t, 127)
            cv = jnp.full((16,), c, jnp.int32)
            lv = jnp.full((16,), l, jnp.int32)
            v0 = buf[t, pl.ds(0, 16)]
            plsc.store_scatter(slab, [a_lo, cv, r_v, lv], v0)
            v1 = buf[t, pl.ds(16, 16)]
            plsc.store_scatter(slab, [a_hi, cv, r_v, lv], v1)

    def write_slab(u):
        s = u // 4
        q = u % 4
        pltpu.sync_copy(
            slab.at[:, :, :, pl.ds(0, 128)],
            out_hbm.at[s, slice(None), pl.ds(q * 8, 8)])

    load_idx(base_u, idx0)
    g0 = start_gather(idx0, buf0, semg0)

    def wait_gather(idx_v, buf, semg):
        pltpu.make_async_copy(table_hbm.at[idx_v], buf, semg).wait()

    def pair(i, carry):
        u0 = base_u + 2 * i
        load_idx(u0 + 1, idx1)
        start_gather(idx1, buf1, semg1)
        wait_gather(idx0, buf0, semg0)
        transpose_unit(buf0)
        write_slab(u0)
        load_idx(u0 + 2, idx0)
        start_gather(idx0, buf0, semg0)
        wait_gather(idx1, buf1, semg1)
        transpose_unit(buf1)
        write_slab(u0 + 1)
        return carry

    lax.fori_loop(0, (UPW - 1) // 2, pair, 0)

    u_last = base_u + UPW - 1
    wait_gather(idx0, buf0, semg0)
    transpose_unit(buf0)
    write_slab(u_last)


def kernel(x, table):
    xf = x.T.reshape(-1)
    mesh = plsc.VectorSubcoreMesh(core_axis_name="c", subcore_axis_name="s")
    run = functools.partial(
        pl.kernel,
        mesh=mesh,
        out_type=jax.ShapeDtypeStruct((200, 4, 32, 8, 128), jnp.float32),
        scratch_types=[
            pltpu.VMEM((TOK,), jnp.int32),
            pltpu.VMEM((TOK,), jnp.int32),
            pltpu.VMEM((TOK, EMB), jnp.float32),
            pltpu.VMEM((TOK, EMB), jnp.float32),
            pltpu.VMEM((4, 8, 8, 129), jnp.float32),
            pltpu.SemaphoreType.DMA,
            pltpu.SemaphoreType.DMA,
        ],
        compiler_params=pltpu.CompilerParams(
            use_tc_tiling_on_sc=False, needs_layout_passes=False),
    )(_emb_body)
    out5d = run(xf, table)
    return out5d.transpose(2, 4, 0, 1, 3).reshape(4096, 200, 32)
